# baseline (speedup 1.0000x reference)
import jax
import jax.numpy as jnp
from jax import lax
from jax.experimental import pallas as pl
from jax.experimental.pallas import tpu as pltpu
from jax.experimental.pallas import tpu_sc as plsc

_TOP_P = 0.9
_TOP_K = 50
_B = 128
_V = 100000
_L = 16
_NW = 32
_RPW = _B // _NW
_NSUB = 128
_NACC = _NSUB // _L
_VPAD = 100096
_NVREG = _VPAD // _L
_NGRP = _NVREG // _NACC
_CHUNK_GRPS = (196, 196, 196, 193)
_CHUNK_ELEMS = tuple(g * _NSUB for g in _CHUNK_GRPS)
_TAIL_BASE = sum(_CHUNK_ELEMS)
_TAIL = _V - _TAIL_BASE
_P2G = 16
_CAP = 384
_FMIN = float(jnp.finfo(jnp.float32).min)


def _iota16():
    return lax.iota(jnp.int32, 16)


def _worker_id():
    return lax.axis_index("s") * 2 + lax.axis_index("c")


def _make_mesh():
    return plsc.VectorSubcoreMesh(core_axis_name="c", subcore_axis_name="s",
                                  num_cores=2, num_subcores=16)


def _f32_key(v):
    u = lax.bitcast_convert_type(v, jnp.uint32)
    neg = (u >> 31) == 1
    return jnp.where(neg, ~u, u | jnp.uint32(0x80000000))


def _key_to_f32(k):
    neg = (k >> 31) == 0
    u = jnp.where(neg, ~k, k ^ jnp.uint32(0x80000000))
    return lax.bitcast_convert_type(u, jnp.float32)


def _kth_largest_of_vregs(vregs, k):
    keys = [_f32_key(v) for v in vregs]

    def bs_body(_, carry):
        lo, hi = carry
        mid = lo + ((hi - lo + jnp.uint32(1)) >> 1)
        acc = jnp.zeros((16,), jnp.int32)
        for kv in keys:
            acc = acc + (kv >= mid).astype(jnp.int32)
        cnt = jnp.sum(acc)
        ge = cnt >= k
        return (jnp.where(ge, mid, lo),
                jnp.where(ge, hi, mid - jnp.uint32(1)))

    lo, _ = lax.fori_loop(
        0, 32, bs_body, (jnp.uint32(0), jnp.uint32(0xFFFFFFFE)))
    return _key_to_f32((jnp.zeros((16,), jnp.uint32) + lo))[0]


def _sc_kernel_body(logits_hbm, conf_hbm, x0_hbm, row_v, tail_v, cand_v,
                    cidx_v, ej_v, pj_v, rank_v, g_v, res_f_v, res_i_v,
                    sval_s, sprob_s, sems):
    wid = _worker_id()
    it = _iota16()
    fmin16 = jnp.full((16,), _FMIN, jnp.float32)

    for j in range(_V // _L, _NVREG):
        row_v[pl.ds(j * 16, 16)] = fmin16

    conf_vec = jnp.zeros((16,), jnp.float32)
    x0_vec = jnp.zeros((16,), jnp.int32)

    def fire_row_dma(row):
        copies = []
        base = 0
        for c in range(4):
            n = _CHUNK_ELEMS[c]
            copies.append(pltpu.async_copy(
                logits_hbm.at[row, pl.ds(base, n)],
                row_v.at[pl.ds(base, n)], sems.at[c]))
            base += n
        copies.append(pltpu.async_copy(
            logits_hbm.at[row, pl.ds(_TAIL_BASE, _TAIL)], tail_v, sems.at[4]))
        return copies

    copies = fire_row_dma(wid * _RPW)

    for r in range(_RPW):
        row = wid * _RPW + r

        accs = tuple(fmin16 for _ in range(_NACC))
        gbase = 0
        for c in range(4):
            copies[c].wait()

            def p1_body(j, accs):
                base = j * (_NACC * 16)
                return tuple(
                    jnp.maximum(accs[g], row_v[pl.ds(base + g * 16, 16)])
                    for g in range(_NACC))

            accs = plsc.parallel_loop(
                gbase, gbase + _CHUNK_GRPS[c], unroll=2, carry=accs)(p1_body)
            gbase += _CHUNK_GRPS[c]

        copies[4].wait()
        accs = list(accs)
        for i in range(_TAIL // 16):
            tv = tail_v[pl.ds(i * 16, 16)]
            row_v[pl.ds(_TAIL_BASE + i * 16, 16)] = tv
            accs[i % _NACC] = jnp.maximum(accs[i % _NACC], tv)
        accs = tuple(accs)

        allmax = accs[0]
        for g in range(1, _NACC):
            allmax = jnp.maximum(allmax, accs[g])
        lmax = jnp.max(allmax)
        tau = _kth_largest_of_vregs(list(accs), _TOP_K)

        for j in range((_CAP + 16) // 16):
            cand_v[pl.ds(j * 16, 16)] = fmin16

        def compact_group(jg, off):
            base = jg * (_P2G * 16)
            for u in range(_P2G):
                x = row_v[pl.ds(base + u * 16, 16)]
                mask = x >= tau
                cnt16 = plsc.all_reduce_population_count(mask)[0]
                plsc.store_compressed(
                    cand_v.at[pl.ds(off, 16)], x, mask=mask)
                plsc.store_compressed(
                    cidx_v.at[pl.ds(off, 16)], base + u * 16 + it, mask=mask)
                off = jnp.minimum(off + cnt16, _CAP)
            return off

        def p2_body(jg, carry):
            off, z0, z1, z2, z3, prev_nany = carry
            off = lax.cond(prev_nany > 0,
                           lambda o: compact_group(jg - 1, o),
                           lambda o: o, off)
            base = jg * (_P2G * 16)
            xs = [row_v[pl.ds(base + u * 16, 16)] for u in range(_P2G)]
            masks = [x >= tau for x in xs]
            for u in range(0, _P2G, 4):
                z0 = z0 + jnp.exp(xs[u] - lmax)
                z1 = z1 + jnp.exp(xs[u + 1] - lmax)
                z2 = z2 + jnp.exp(xs[u + 2] - lmax)
                z3 = z3 + jnp.exp(xs[u + 3] - lmax)
            while len(masks) > 1:
                masks = [masks[2 * i] | masks[2 * i + 1]
                         for i in range(len(masks) // 2)]
            nany = plsc.all_reduce_population_count(masks[0])[0]
            return off, z0, z1, z2, z3, nany

        zi = jnp.zeros((16,), jnp.float32)
        off, z0, z1, z2, z3, last_nany = plsc.parallel_loop(
            0, _NVREG // _P2G,
            carry=(jnp.int32(0), zi, zi, zi, zi, jnp.int32(0)))(
                lambda jg, c: p2_body(jg, c))
        off = lax.cond(last_nany > 0,
                       lambda o: compact_group(_NVREG // _P2G - 1, o),
                       lambda o: o, off)
        z_all = jnp.sum((z0 + z1) + (z2 + z3))
        cnt = jnp.minimum(off, _CAP)
        nvec = (cnt + 15) >> 4

        if r + 1 < _RPW:
            copies = fire_row_dma(row + 1)

        def prol_body(o, _):
            ci = cand_v[pl.ds(o * 16, 16)]
            e = jnp.exp(ci - lmax)
            p = e / z_all
            ej_v[pl.ds(o * 16, 16)] = e
            pj_v[pl.ds(o * 16, 16)] = p
            for l in range(16):
                sval_s[o * 16 + l] = ci[l]
                sprob_s[o * 16 + l] = p[l]
            return 0

        lax.fori_loop(0, nvec, prol_body, 0)

        def ro_body(o, _):
            ci = cand_v[pl.ds(o * 16, 16)]
            ipos = o * 16 + it

            def rj_step(j, rk, g):
                cj = sval_s[j]
                pj = sprob_s[j]
                m = (cj > ci) | ((cj == ci) & (j < ipos))
                return rk + m.astype(jnp.int32), g + jnp.where(m, pj, 0.0)

            def rj_body4(j4, c):
                rk, g = c
                for dj in range(4):
                    rk, g = rj_step(j4 * 4 + dj, rk, g)
                return rk, g

            def rj_body(j, c):
                return rj_step(j, *c)

            cnt4 = cnt & ~jnp.int32(3)
            rk, g = lax.fori_loop(
                0, cnt4 >> 2, rj_body4,
                (jnp.zeros((16,), jnp.int32), jnp.zeros((16,), jnp.float32)))
            rk, g = lax.fori_loop(cnt4, cnt, rj_body, (rk, g))
            rank_v[pl.ds(o * 16, 16)] = rk
            g_v[pl.ds(o * 16, 16)] = g
            return 0

        lax.fori_loop(0, nvec, ro_body, 0)

        def vk_body(o, acc):
            ci = cand_v[pl.ds(o * 16, 16)]
            rk = rank_v[pl.ds(o * 16, 16)]
            return jnp.maximum(
                acc, jnp.max(jnp.where(rk == _TOP_K - 1, ci, _FMIN)))

        vk = lax.fori_loop(0, nvec, vk_body, jnp.float32(_FMIN))

        def stats_body(o, carry):
            m_acc, cge_acc = carry
            ci = cand_v[pl.ds(o * 16, 16)]
            rk = rank_v[pl.ds(o * 16, 16)]
            g = g_v[pl.ds(o * 16, 16)]
            valid = (o * 16 + it) < cnt
            kept = ((rk == 0) | (g <= _TOP_P)) & valid
            cge = (ci >= vk) & valid
            return (m_acc + kept.astype(jnp.int32),
                    cge_acc + cge.astype(jnp.int32))

        m_acc, cge_acc = lax.fori_loop(
            0, nvec, stats_body,
            (jnp.zeros((16,), jnp.int32), jnp.zeros((16,), jnp.int32)))
        n_s = jnp.minimum(jnp.sum(m_acc), jnp.sum(cge_acc))

        def fin_body(o, carry):
            zs_acc, x0_acc = carry
            ci = cand_v[pl.ds(o * 16, 16)]
            rk = rank_v[pl.ds(o * 16, 16)]
            ej = ej_v[pl.ds(o * 16, 16)]
            ix = cidx_v[pl.ds(o * 16, 16)]
            sel = rk < n_s
            zs_acc = zs_acc + jnp.where(sel, ej, 0.0)
            x0_acc = jnp.maximum(
                x0_acc, jnp.where(sel & (ci == lmax), ix, -1))
            return zs_acc, x0_acc

        zs_acc, x0_acc = lax.fori_loop(
            0, nvec, fin_body,
            (jnp.zeros((16,), jnp.float32),
             jnp.full((16,), -1, jnp.int32)))
        recip = 1.0 / (jnp.zeros((16,), jnp.float32) + jnp.sum(zs_acc))
        x0 = jnp.max(x0_acc)

        conf_vec = jnp.where(it == r, recip, conf_vec)
        x0_vec = jnp.where(it == r, x0, x0_vec)

    res_f_v[...] = conf_vec
    res_i_v[...] = x0_vec
    pltpu.sync_copy(res_f_v, conf_hbm.at[wid])
    pltpu.sync_copy(res_i_v, x0_hbm.at[wid])


@jax.jit
def kernel(logits):
    mesh = _make_mesh()
    conf_out, x0_out = pl.kernel(
        _sc_kernel_body,
        out_type=[
            jax.ShapeDtypeStruct((_NW, 16), jnp.float32),
            jax.ShapeDtypeStruct((_NW, 16), jnp.int32),
        ],
        mesh=mesh,
        compiler_params=pltpu.CompilerParams(needs_layout_passes=False,
                                             use_tc_tiling_on_sc=True),
        scratch_types=[
            pltpu.VMEM((_VPAD,), jnp.float32),
            pltpu.VMEM((_TAIL,), jnp.float32),
            pltpu.VMEM((_CAP + 16,), jnp.float32),
            pltpu.VMEM((_CAP + 16,), jnp.int32),
            pltpu.VMEM((_CAP,), jnp.float32),
            pltpu.VMEM((_CAP,), jnp.float32),
            pltpu.VMEM((_CAP,), jnp.int32),
            pltpu.VMEM((_CAP,), jnp.float32),
            pltpu.VMEM((16,), jnp.float32),
            pltpu.VMEM((16,), jnp.int32),
            pltpu.SMEM((_CAP,), jnp.float32),
            pltpu.SMEM((_CAP,), jnp.float32),
            pltpu.SemaphoreType.DMA((5,)),
        ],
    )(logits)
    conf = conf_out[:, :_RPW].reshape(_B)
    x0 = x0_out[:, :_RPW].reshape(_B).astype(jnp.int64)
    return conf, x0, conf


# speedup vs baseline: 1.1638x; 1.1638x over previous
import jax
import jax.numpy as jnp
from jax import lax
from jax.experimental import pallas as pl
from jax.experimental.pallas import tpu as pltpu
from jax.experimental.pallas import tpu_sc as plsc

_TOP_P = 0.9
_TOP_K = 50
_B = 128
_V = 100000
_L = 16
_NW = 32
_RPW = _B // _NW
_NSUB = 128
_NACC = _NSUB // _L
_VPAD = 100096
_NVREG = _VPAD // _L
_NGRP = _NVREG // _NACC
_CHUNK_GRPS = (196, 196, 196, 193)
_CHUNK_ELEMS = tuple(g * _NSUB for g in _CHUNK_GRPS)
_TAIL_BASE = sum(_CHUNK_ELEMS)
_TAIL = _V - _TAIL_BASE
_P2G = 16
_CAP = 384
_FMIN = float(jnp.finfo(jnp.float32).min)


def _iota16():
    return lax.iota(jnp.int32, 16)


def _worker_id():
    return lax.axis_index("s") * 2 + lax.axis_index("c")


def _make_mesh():
    return plsc.VectorSubcoreMesh(core_axis_name="c", subcore_axis_name="s",
                                  num_cores=2, num_subcores=16)


def _f32_key(v):
    u = lax.bitcast_convert_type(v, jnp.uint32)
    neg = (u >> 31) == 1
    return jnp.where(neg, ~u, u | jnp.uint32(0x80000000))


def _key_to_f32(k):
    neg = (k >> 31) == 0
    u = jnp.where(neg, ~k, k ^ jnp.uint32(0x80000000))
    return lax.bitcast_convert_type(u, jnp.float32)


def _kth_largest_of_vregs(vregs, k):
    keys = [_f32_key(v) for v in vregs]

    def bs_body(_, carry):
        lo, hi = carry
        mid = lo + ((hi - lo + jnp.uint32(1)) >> 1)
        acc = jnp.zeros((16,), jnp.int32)
        for kv in keys:
            acc = acc + (kv >= mid).astype(jnp.int32)
        cnt = jnp.sum(acc)
        ge = cnt >= k
        return (jnp.where(ge, mid, lo),
                jnp.where(ge, hi, mid - jnp.uint32(1)))

    lo, _ = lax.fori_loop(
        0, 32, bs_body, (jnp.uint32(0), jnp.uint32(0xFFFFFFFE)))
    return _key_to_f32((jnp.zeros((16,), jnp.uint32) + lo))[0]


def _sc_kernel_body(logits_hbm, conf_hbm, x0_hbm, row_v, tail_v, cand_v,
                    cidx_v, ej_v, pj_v, rank_v, g_v, res_f_v, res_i_v,
                    sval_s, sprob_s, sems):
    wid = _worker_id()
    it = _iota16()
    fmin16 = jnp.full((16,), _FMIN, jnp.float32)

    for j in range(_V // _L, _NVREG):
        row_v[pl.ds(j * 16, 16)] = fmin16

    conf_vec = jnp.zeros((16,), jnp.float32)
    x0_vec = jnp.zeros((16,), jnp.int32)

    def fire_row_dma(row):
        copies = []
        base = 0
        for c in range(4):
            n = _CHUNK_ELEMS[c]
            copies.append(pltpu.async_copy(
                logits_hbm.at[row, pl.ds(base, n)],
                row_v.at[pl.ds(base, n)], sems.at[c]))
            base += n
        copies.append(pltpu.async_copy(
            logits_hbm.at[row, pl.ds(_TAIL_BASE, _TAIL)], tail_v, sems.at[4]))
        return copies

    copies = fire_row_dma(wid * _RPW)

    for r in range(_RPW):
        row = wid * _RPW + r

        accs = tuple(fmin16 for _ in range(_NACC))
        gbase = 0
        for c in range(4):
            copies[c].wait()

            def p1_body(j, accs):
                base = j * (_NACC * 16)
                return tuple(
                    jnp.maximum(accs[g], row_v[pl.ds(base + g * 16, 16)])
                    for g in range(_NACC))

            accs = plsc.parallel_loop(
                gbase, gbase + _CHUNK_GRPS[c], unroll=2, carry=accs)(p1_body)
            gbase += _CHUNK_GRPS[c]

        copies[4].wait()
        accs = list(accs)
        for i in range(_TAIL // 16):
            tv = tail_v[pl.ds(i * 16, 16)]
            row_v[pl.ds(_TAIL_BASE + i * 16, 16)] = tv
            accs[i % _NACC] = jnp.maximum(accs[i % _NACC], tv)
        accs = tuple(accs)

        allmax = accs[0]
        for g in range(1, _NACC):
            allmax = jnp.maximum(allmax, accs[g])
        lmax = jnp.max(allmax)
        tau = _kth_largest_of_vregs(list(accs), _TOP_K)

        for j in range((_CAP + 16) // 16):
            cand_v[pl.ds(j * 16, 16)] = fmin16

        def p2_body(jg, carry):
            off, z0, z1, z2, z3 = carry
            base = jg * (_P2G * 16)
            xs = [row_v[pl.ds(base + u * 16, 16)] for u in range(_P2G)]
            masks = [x >= tau for x in xs]
            for u in range(0, _P2G, 4):
                z0 = z0 + jnp.exp(xs[u] - lmax)
                z1 = z1 + jnp.exp(xs[u + 1] - lmax)
                z2 = z2 + jnp.exp(xs[u + 2] - lmax)
                z3 = z3 + jnp.exp(xs[u + 3] - lmax)
            anym = masks[0]
            for m in masks[1:]:
                anym = anym | m
            nany = plsc.all_reduce_population_count(anym)[0]

            def slow(off):
                for u in range(_P2G):
                    mask = masks[u]
                    cnt16 = plsc.all_reduce_population_count(mask)[0]
                    plsc.store_compressed(
                        cand_v.at[pl.ds(off, 16)], xs[u], mask=mask)
                    plsc.store_compressed(
                        cidx_v.at[pl.ds(off, 16)],
                        base + u * 16 + it, mask=mask)
                    off = jnp.minimum(off + cnt16, _CAP)
                return off

            off = lax.cond(nany > 0, slow, lambda o: o, off)
            return off, z0, z1, z2, z3

        zi = jnp.zeros((16,), jnp.float32)
        off, z0, z1, z2, z3 = plsc.parallel_loop(
            0, _NVREG // _P2G, carry=(jnp.int32(0), zi, zi, zi, zi))(
                lambda jg, c: p2_body(jg, c))
        z_all = jnp.sum((z0 + z1) + (z2 + z3))
        cnt = jnp.minimum(off, _CAP)
        nvec = (cnt + 15) >> 4

        if r + 1 < _RPW:
            copies = fire_row_dma(row + 1)

        def prol_body(o, _):
            ci = cand_v[pl.ds(o * 16, 16)]
            e = jnp.exp(ci - lmax)
            p = e / z_all
            ej_v[pl.ds(o * 16, 16)] = e
            pj_v[pl.ds(o * 16, 16)] = p
            for l in range(16):
                sval_s[o * 16 + l] = ci[l]
                sprob_s[o * 16 + l] = p[l]
            return 0

        lax.fori_loop(0, nvec, prol_body, 0)

        def ro_body(o, _):
            ci = cand_v[pl.ds(o * 16, 16)]
            ipos = o * 16 + it

            def rj_step(j, rk, g):
                cj = sval_s[j]
                pj = sprob_s[j]
                m = (cj > ci) | ((cj == ci) & (j < ipos))
                return rk + m.astype(jnp.int32), g + jnp.where(m, pj, 0.0)

            def rj_body4(j4, c):
                rk, g = c
                for dj in range(4):
                    rk, g = rj_step(j4 * 4 + dj, rk, g)
                return rk, g

            def rj_body(j, c):
                return rj_step(j, *c)

            cnt4 = cnt & ~jnp.int32(3)
            rk, g = lax.fori_loop(
                0, cnt4 >> 2, rj_body4,
                (jnp.zeros((16,), jnp.int32), jnp.zeros((16,), jnp.float32)))
            rk, g = lax.fori_loop(cnt4, cnt, rj_body, (rk, g))
            rank_v[pl.ds(o * 16, 16)] = rk
            g_v[pl.ds(o * 16, 16)] = g
            return 0

        lax.fori_loop(0, nvec, ro_body, 0)

        def vk_body(o, acc):
            ci = cand_v[pl.ds(o * 16, 16)]
            rk = rank_v[pl.ds(o * 16, 16)]
            return jnp.maximum(
                acc, jnp.max(jnp.where(rk == _TOP_K - 1, ci, _FMIN)))

        vk = lax.fori_loop(0, nvec, vk_body, jnp.float32(_FMIN))

        def stats_body(o, carry):
            m_acc, cge_acc = carry
            ci = cand_v[pl.ds(o * 16, 16)]
            rk = rank_v[pl.ds(o * 16, 16)]
            g = g_v[pl.ds(o * 16, 16)]
            valid = (o * 16 + it) < cnt
            kept = ((rk == 0) | (g <= _TOP_P)) & valid
            cge = (ci >= vk) & valid
            return (m_acc + kept.astype(jnp.int32),
                    cge_acc + cge.astype(jnp.int32))

        m_acc, cge_acc = lax.fori_loop(
            0, nvec, stats_body,
            (jnp.zeros((16,), jnp.int32), jnp.zeros((16,), jnp.int32)))
        n_s = jnp.minimum(jnp.sum(m_acc), jnp.sum(cge_acc))

        def fin_body(o, carry):
            zs_acc, x0_acc = carry
            ci = cand_v[pl.ds(o * 16, 16)]
            rk = rank_v[pl.ds(o * 16, 16)]
            ej = ej_v[pl.ds(o * 16, 16)]
            ix = cidx_v[pl.ds(o * 16, 16)]
            sel = rk < n_s
            zs_acc = zs_acc + jnp.where(sel, ej, 0.0)
            x0_acc = jnp.maximum(
                x0_acc, jnp.where(sel & (ci == lmax), ix, -1))
            return zs_acc, x0_acc

        zs_acc, x0_acc = lax.fori_loop(
            0, nvec, fin_body,
            (jnp.zeros((16,), jnp.float32),
             jnp.full((16,), -1, jnp.int32)))
        recip = 1.0 / (jnp.zeros((16,), jnp.float32) + jnp.sum(zs_acc))
        x0 = jnp.max(x0_acc)

        conf_vec = jnp.where(it == r, recip, conf_vec)
        x0_vec = jnp.where(it == r, x0, x0_vec)

    res_f_v[...] = conf_vec
    res_i_v[...] = x0_vec
    pltpu.sync_copy(res_f_v, conf_hbm.at[wid])
    pltpu.sync_copy(res_i_v, x0_hbm.at[wid])


@jax.jit
def kernel(logits):
    mesh = _make_mesh()
    conf_out, x0_out = pl.kernel(
        _sc_kernel_body,
        out_type=[
            jax.ShapeDtypeStruct((_NW, 16), jnp.float32),
            jax.ShapeDtypeStruct((_NW, 16), jnp.int32),
        ],
        mesh=mesh,
        compiler_params=pltpu.CompilerParams(needs_layout_passes=False,
                                             use_tc_tiling_on_sc=True),
        scratch_types=[
            pltpu.VMEM((_VPAD,), jnp.float32),
            pltpu.VMEM((_TAIL,), jnp.float32),
            pltpu.VMEM((_CAP + 16,), jnp.float32),
            pltpu.VMEM((_CAP + 16,), jnp.int32),
            pltpu.VMEM((_CAP,), jnp.float32),
            pltpu.VMEM((_CAP,), jnp.float32),
            pltpu.VMEM((_CAP,), jnp.int32),
            pltpu.VMEM((_CAP,), jnp.float32),
            pltpu.VMEM((16,), jnp.float32),
            pltpu.VMEM((16,), jnp.int32),
            pltpu.SMEM((_CAP,), jnp.float32),
            pltpu.SMEM((_CAP,), jnp.float32),
            pltpu.SemaphoreType.DMA((5,)),
        ],
    )(logits)
    conf = conf_out[:, :_RPW].reshape(_B)
    x0 = x0_out[:, :_RPW].reshape(_B).astype(jnp.int64)
    return conf, x0, conf
